# baseline (speedup 1.0000x reference)
import functools

import jax
import jax.numpy as jnp
from jax import lax
from jax.experimental import pallas as pl
from jax.experimental.pallas import tpu as pltpu
from jax.experimental.pallas import tpu_sc as plsc

B = 4
S = 2048
D = 2048
NC = 2
NS = 16
NW = NC * NS
PRW = S // NW
C = 8
NP = PRW // C
NT = NP * B
L = 16
UNROLL = 8
NBUF = 3
TB = 2

_mesh = plsc.VectorSubcoreMesh(core_axis_name="c", subcore_axis_name="s")


@functools.partial(
    pl.kernel,
    mesh=_mesh,
    out_type=jax.ShapeDtypeStruct((B, S, D), jnp.float32),
    scratch_types=[
        pltpu.VMEM_SHARED((NS, NBUF, C, D), jnp.float32),
        pltpu.VMEM((TB, C, D), jnp.float32),
        pltpu.VMEM((2, C, D), jnp.float32),
        pltpu.SemaphoreType.DMA((2,)),
        pltpu.SemaphoreType.DMA((NBUF,)),
        pltpu.SemaphoreType.DMA((TB,)),
        pltpu.SemaphoreType.DMA((TB,)),
        pltpu.SemaphoreType.DMA((NBUF,)),
    ],
)
def _sc_add(x_hbm, pos_hbm, out_hbm, stage, tbuf, pos_v,
            pld_sem, ld_sem, xi_sem, xo_sem, st_sem):
    sid = lax.axis_index("s")
    wid = sid * NC + lax.axis_index("c")
    pos_row0 = wid * PRW

    def start_pos_load(p, q):
        r = pos_row0 + p * C
        pltpu.async_copy(pos_hbm.at[pl.ds(r, C), :], pos_v.at[q], pld_sem.at[q])

    def wait_pos_load(q):
        pltpu.make_async_copy(
            pos_hbm.at[pl.ds(0, C), :], pos_v.at[q], pld_sem.at[q]).wait()

    def start_load(t, s):
        p = t // B
        b = t % B
        r = pos_row0 + p * C
        pltpu.async_copy(
            x_hbm.at[b, pl.ds(r, C), :], stage.at[sid, s], ld_sem.at[s])

    def wait_load(s):
        pltpu.make_async_copy(
            x_hbm.at[0, pl.ds(0, C), :], stage.at[sid, s], ld_sem.at[s]).wait()

    def start_xin(s, u):
        pltpu.async_copy(stage.at[sid, s], tbuf.at[u], xi_sem.at[u])

    def wait_xin(u):
        pltpu.make_async_copy(
            stage.at[sid, 0], tbuf.at[u], xi_sem.at[u]).wait()

    def start_xout(s, u):
        pltpu.async_copy(tbuf.at[u], stage.at[sid, s], xo_sem.at[u])

    def wait_xout(u):
        pltpu.make_async_copy(
            tbuf.at[u], stage.at[sid, 0], xo_sem.at[u]).wait()

    def start_store(t, s):
        p = t // B
        b = t % B
        r = pos_row0 + p * C
        pltpu.async_copy(
            stage.at[sid, s], out_hbm.at[b, pl.ds(r, C), :], st_sem.at[s])

    def wait_store(s):
        pltpu.make_async_copy(
            stage.at[sid, s], out_hbm.at[0, pl.ds(0, C), :], st_sem.at[s]).wait()

    def add(u, q):
        def add_body(k, carry):
            base = k * (L * UNROLL)
            for row in range(C):
                for j in range(UNROLL):
                    sl = pl.ds(base + j * L, L)
                    tbuf[u, row, sl] = tbuf[u, row, sl] + pos_v[q, row, sl]
            return carry

        lax.fori_loop(0, D // (L * UNROLL), add_body, 0)

    start_pos_load(0, 0)
    start_load(0, 0)

    def chunk_body(t, carry):
        s = lax.rem(t, NBUF)
        sn = lax.rem(t + 1, NBUF)
        u = lax.rem(t, TB)
        p = t // B
        b = lax.rem(t, B)
        q = lax.rem(p, 2)

        @pl.when(t + 1 < NT)
        def _():
            @pl.when(t + 1 >= NBUF)
            def _():
                wait_store(sn)
            start_load(t + 1, sn)

        @pl.when(b == 0)
        def _():
            wait_pos_load(q)

        wait_load(s)
        start_xin(s, u)

        @pl.when(t >= 1)
        def _():
            wait_xout(lax.rem(t - 1, TB))
            start_store(t - 1, lax.rem(t - 1, NBUF))

        wait_xin(u)
        add(u, q)
        start_xout(s, u)

        @pl.when((b == 0) & (p + 1 < NP))
        def _():
            start_pos_load(p + 1, 1 - q)

        return carry

    lax.fori_loop(0, NT, chunk_body, 0)
    wait_xout((NT - 1) % TB)
    start_store(NT - 1, (NT - 1) % NBUF)
    for t in range(NT - NBUF, NT):
        wait_store(t % NBUF)


def kernel(x, pos_emb):
    return _sc_add(x, pos_emb)


# speedup vs baseline: 1.3075x; 1.3075x over previous
import functools

import jax
import jax.numpy as jnp
from jax import lax
from jax.experimental import pallas as pl
from jax.experimental.pallas import tpu as pltpu
from jax.experimental.pallas import tpu_sc as plsc

B = 4
S = 2048
D = 2048
NC = 2
NS = 16
NW = NC * NS
PRW = S // NW
C = 4
NCHUNK = PRW // C
L = 16
UNROLL = 8
NBUF = 3

_mesh = plsc.VectorSubcoreMesh(core_axis_name="c", subcore_axis_name="s")


@functools.partial(
    pl.kernel,
    mesh=_mesh,
    out_type=jax.ShapeDtypeStruct((B, S, D), jnp.float32),
    scratch_types=[
        pltpu.VMEM((NBUF, C, D), jnp.float32),
        pltpu.VMEM((NBUF, B, C, D), jnp.float32),
        pltpu.SemaphoreType.DMA((NBUF,)),
        pltpu.SemaphoreType.DMA((NBUF,)),
    ],
)
def _sc_add(x_hbm, pos_hbm, out_hbm, pos_v, xb_v, ld_sem, st_sem):
    wid = lax.axis_index("s") * NC + lax.axis_index("c")
    pos_row0 = wid * PRW

    def start_load(i, s):
        r = pos_row0 + i * C
        pltpu.async_copy(pos_hbm.at[pl.ds(r, C), :], pos_v.at[s], ld_sem.at[s])
        for b in range(B):
            pltpu.async_copy(
                x_hbm.at[b, pl.ds(r, C), :], xb_v.at[s, b], ld_sem.at[s])

    def wait_load(s):
        pltpu.make_async_copy(
            pos_hbm.at[pl.ds(0, C), :], pos_v.at[s], ld_sem.at[s]).wait()
        for b in range(B):
            pltpu.make_async_copy(
                x_hbm.at[0, pl.ds(0, C), :], xb_v.at[s, b], ld_sem.at[s]).wait()

    def start_store(i, s):
        r = pos_row0 + i * C
        for b in range(B):
            pltpu.async_copy(
                xb_v.at[s, b], out_hbm.at[b, pl.ds(r, C), :], st_sem.at[s])

    def wait_store(s):
        for b in range(B):
            pltpu.make_async_copy(
                xb_v.at[s, b], out_hbm.at[0, pl.ds(0, C), :],
                st_sem.at[s]).wait()

    def compute(s):
        def add_body(k, carry):
            base = k * (L * UNROLL)
            for row in range(C):
                for j in range(UNROLL):
                    sl = pl.ds(base + j * L, L)
                    pv = pos_v[s, row, sl]
                    for b in range(B):
                        xb_v[s, b, row, sl] = xb_v[s, b, row, sl] + pv
            return carry

        lax.fori_loop(0, D // (L * UNROLL), add_body, 0)

    start_load(0, 0)

    def chunk_body(i, carry):
        s = lax.rem(i, NBUF)
        sn = lax.rem(i + 1, NBUF)

        @pl.when(i >= NBUF - 1)
        def _():
            wait_store(sn)

        @pl.when(i + 1 < NCHUNK)
        def _():
            start_load(i + 1, sn)

        wait_load(s)
        compute(s)
        start_store(i, s)
        return carry

    lax.fori_loop(0, NCHUNK, chunk_body, 0)
    wait_store((NCHUNK - 2) % NBUF)
    wait_store((NCHUNK - 1) % NBUF)


def kernel(x, pos_emb):
    return _sc_add(x, pos_emb)
